# baseline (speedup 1.0000x reference)
import functools

import jax
import jax.numpy as jnp
from jax import lax
from jax.experimental import pallas as pl
from jax.experimental.pallas import tpu as pltpu
from jax.experimental.pallas import tpu_sc as plsc

N = 10000
F = 128
E = 320000
C = 2
NPAD = 10240
TRASH = N

NCORES = 2
NSUB = 16
NW = NCORES * NSUB
EPT = E // NW
K = 80
NCH = EPT // K
RPT = NPAD // NSUB


def _evolve_body(xp_ref, p_ref, iw_ref, wih_ref, whh_ref, bih_ref, bhh_ref,
                 w_out_ref, s_scr, bm_scr, xt_scr):
    p = p_ref[...]
    pnorm = jnp.sqrt(jnp.sum(p * p))
    s = lax.dot_general(p, xp_ref[...], (((1,), (1,)), ((), ())),
                        preferred_element_type=jnp.float32)
    col = lax.broadcasted_iota(jnp.int32, (1, NPAD), 1)
    s2 = jnp.reshape(jnp.where(col < N, s, -jnp.inf), (NPAD // F, F))
    s_scr[...] = s2
    bm_scr[...] = jnp.max(s2, axis=1, keepdims=True)

    riota = lax.broadcasted_iota(jnp.int32, (NPAD // F, 1), 0)
    liota = lax.broadcasted_iota(jnp.int32, (1, F), 1)
    ninf = jnp.float32(-jnp.inf)

    def sel_body(i, _):
        bm = bm_scr[...]
        m = jnp.max(bm)
        r = jnp.min(jnp.where(bm == m, riota, NPAD // F))
        row = s_scr[pl.ds(r, 1), :]
        cc = jnp.min(jnp.where(row == m, liota, F))
        nrow = jnp.where(liota == cc, ninf, row)
        s_scr[pl.ds(r, 1), :] = nrow
        bm_scr[...] = jnp.where(riota == r, jnp.max(nrow), bm)
        idx = r * F + cc
        xrow = xp_ref[pl.ds(idx, 1), :]
        xt_scr[pl.ds(i, 1), :] = xrow * jnp.tanh(m / pnorm)
        return 0

    lax.fori_loop(0, F, sel_body, 0)

    xt = xt_scr[...]
    gi = lax.dot_general(xt, wih_ref[...], (((1,), (1,)), ((), ())),
                         preferred_element_type=jnp.float32) + bih_ref[...]
    gh = lax.dot_general(iw_ref[...], whh_ref[...], (((1,), (1,)), ((), ())),
                         preferred_element_type=jnp.float32) + bhh_ref[...]
    r = jax.nn.sigmoid(gi[:, :F] + gh[:, :F])
    z = jax.nn.sigmoid(gi[:, F:2 * F] + gh[:, F:2 * F])
    ng = jnp.tanh(gi[:, 2 * F:] + r * gh[:, 2 * F:])
    w_out_ref[...] = (1.0 - z) * ng + z * iw_ref[...]


def _evolve(xp, p2, init_w, W_ih, W_hh, b_ih2, b_hh2):
    return pl.pallas_call(
        _evolve_body,
        out_shape=jax.ShapeDtypeStruct((F, F), jnp.float32),
        scratch_shapes=[
            pltpu.VMEM((NPAD // F, F), jnp.float32),
            pltpu.VMEM((NPAD // F, 1), jnp.float32),
            pltpu.VMEM((F, F), jnp.float32),
        ],
    )(xp, p2, init_w, W_ih, W_hh, b_ih2, b_hh2)


def _deg_body(ei_hbm, out_hbm, acc_sh, srcf, dstf, idxs, ones_v,
              zbuf):
    c = lax.axis_index("c")
    s = lax.axis_index("s")
    wid = s * NCORES + c
    zeros16 = jnp.zeros((16,), jnp.float32)
    ones16 = jnp.ones((16,), jnp.float32)

    def zb(i, _):
        zbuf[i, :] = zeros16
        return 0
    lax.fori_loop(0, RPT, zb, 0)

    def ob(i, _):
        ones_v[i, :] = ones16
        return 0
    lax.fori_loop(0, K, ob, 0)

    pltpu.sync_copy(zbuf, acc_sh.at[pl.ds(s * RPT, RPT)])
    plsc.subcore_barrier()

    base = wid * EPT
    pltpu.sync_copy(ei_hbm.at[0, pl.ds(base, EPT)], srcf)
    pltpu.sync_copy(ei_hbm.at[1, pl.ds(base, EPT)], dstf)

    def remap_chunk(k, _):
        def remap16(j, _):
            off = k * K + j * 16
            sv = srcf[pl.ds(off, 16)]
            dv = dstf[pl.ds(off, 16)]
            idxs[k, pl.ds(j * 16, 16)] = jnp.where(sv == dv, TRASH, dv)
            return 0
        lax.fori_loop(0, K // 16, remap16, 0)
        return 0
    lax.fori_loop(0, NCH, remap_chunk, 0)

    def scat(k, _):
        pltpu.sync_copy(ones_v, acc_sh.at[idxs.at[k]], add=True)
        return 0
    lax.fori_loop(0, NCH, scat, 0)

    plsc.subcore_barrier()
    pltpu.sync_copy(acc_sh.at[pl.ds(s * RPT, RPT)], zbuf)
    pltpu.sync_copy(zbuf, out_hbm.at[c, pl.ds(s * RPT, RPT)])


def _sc_degree(edge_index):
    mesh = plsc.VectorSubcoreMesh(core_axis_name="c", subcore_axis_name="s")
    fn = pl.kernel(
        _deg_body,
        out_type=jax.ShapeDtypeStruct((NCORES, NPAD, 16), jnp.float32),
        mesh=mesh,
        compiler_params=pltpu.CompilerParams(use_tc_tiling_on_sc=False),
        scratch_types=[
            pltpu.VMEM_SHARED((NPAD, 16), jnp.float32),
            pltpu.VMEM((EPT,), jnp.int32),
            pltpu.VMEM((EPT,), jnp.int32),
            pltpu.VMEM((NCH, K), jnp.int32),
            pltpu.VMEM((K, 16), jnp.float32),
            pltpu.VMEM((RPT, 16), jnp.float32),
        ],
    )
    return fn(edge_index)


_BLK = 1024


def _y_body(deg_ref, x_ref, w_ref, y_ref):
    deg = deg_ref[0, :, 0:1] + deg_ref[1, :, 0:1] + 1.0
    dis = 1.0 / jnp.sqrt(deg)
    xw = jnp.dot(x_ref[...], w_ref[...], preferred_element_type=jnp.float32)
    y_ref[...] = dis * xw


def _y(degp, xp, W):
    grid = (NPAD // _BLK,)
    return pl.pallas_call(
        _y_body,
        grid=grid,
        in_specs=[
            pl.BlockSpec((NCORES, _BLK, 16), lambda i: (0, i, 0)),
            pl.BlockSpec((_BLK, F), lambda i: (i, 0)),
            pl.BlockSpec((F, F), lambda i: (0, 0)),
        ],
        out_specs=pl.BlockSpec((_BLK, F), lambda i: (i, 0)),
        out_shape=jax.ShapeDtypeStruct((NPAD, F), jnp.float32),
    )(degp, xp, W)


PH = 5
PEDGE = EPT // PH
PCH = PEDGE // K
NBUF = 3


def _scat_body(y_hbm, ei_hbm, out_hbm, acc_sh, srcf, dstf, idxs,
               rows_v, sem):
    c = lax.axis_index("c")
    s = lax.axis_index("s")
    wid = s * NCORES + c
    zeros16 = jnp.zeros((16,), jnp.float32)

    def zb(i, _):
        for j in range(F // 16):
            rows_v[0, i, pl.ds(j * 16, 16)] = zeros16
        return 0
    lax.fori_loop(0, K, zb, 0)

    for q in range(RPT // K):
        pltpu.sync_copy(rows_v.at[0], acc_sh.at[pl.ds(s * RPT + q * K, K)])
    plsc.subcore_barrier()

    base = wid * EPT

    def phase(ph, _):
        pbase = base + ph * PEDGE
        pltpu.sync_copy(ei_hbm.at[0, pl.ds(pbase, PEDGE)], srcf)
        pltpu.sync_copy(ei_hbm.at[1, pl.ds(pbase, PEDGE)], dstf)

        def remap_chunk(k, _):
            def remap16(j, _):
                off = k * K + j * 16
                sv = srcf[pl.ds(off, 16)]
                dv = dstf[pl.ds(off, 16)]
                idxs[k, pl.ds(j * 16, 16)] = jnp.where(sv == dv, TRASH, dv)
                return 0
            lax.fori_loop(0, K // 16, remap16, 0)
            return 0
        lax.fori_loop(0, PCH, remap_chunk, 0)

        for b in range(NBUF):
            pltpu.async_copy(y_hbm.at[srcf.at[pl.ds(b * K, K)]],
                             rows_v.at[b], sem.at[b])

        def chunk(k, _):
            b = lax.rem(k, NBUF)
            pltpu.make_async_copy(y_hbm.at[srcf.at[pl.ds(k * K, K)]],
                                  rows_v.at[b], sem.at[b]).wait()
            pltpu.sync_copy(rows_v.at[b], acc_sh.at[idxs.at[k]], add=True)

            @pl.when(k + NBUF < PCH)
            def _():
                pltpu.async_copy(
                    y_hbm.at[srcf.at[pl.ds((k + NBUF) * K, K)]],
                    rows_v.at[b], sem.at[b])
            return 0
        lax.fori_loop(0, PCH, chunk, 0)
        return 0
    lax.fori_loop(0, PH, phase, 0)

    plsc.subcore_barrier()
    for q in range(RPT // K):
        r0 = s * RPT + q * K
        pltpu.sync_copy(acc_sh.at[pl.ds(r0, K)], rows_v.at[0])
        pltpu.sync_copy(rows_v.at[0], out_hbm.at[c, pl.ds(r0, K)])


def _sc_scatter(y, edge_index):
    mesh = plsc.VectorSubcoreMesh(core_axis_name="c", subcore_axis_name="s")
    fn = pl.kernel(
        _scat_body,
        out_type=jax.ShapeDtypeStruct((NCORES, NPAD, F), jnp.float32),
        mesh=mesh,
        compiler_params=pltpu.CompilerParams(use_tc_tiling_on_sc=False),
        scratch_types=[
            pltpu.VMEM_SHARED((NPAD, F), jnp.float32),
            pltpu.VMEM((PEDGE,), jnp.int32),
            pltpu.VMEM((PEDGE,), jnp.int32),
            pltpu.VMEM((PCH, K), jnp.int32),
            pltpu.VMEM((NBUF, K, F), jnp.float32),
            pltpu.SemaphoreType.DMA((NBUF,)),
        ],
    )
    return fn(y, edge_index)


def _cls_body(deg_ref, acc_ref, y_ref, wc_ref, bc_ref, o_ref):
    deg = deg_ref[0, :, 0:1] + deg_ref[1, :, 0:1] + 1.0
    dis = 1.0 / jnp.sqrt(deg)
    h = dis * (acc_ref[0] + acc_ref[1] + y_ref[...])
    h = jnp.maximum(h, 0.0)
    o_ref[...] = lax.dot_general(h, wc_ref[...], (((1,), (1,)), ((), ())),
                                 preferred_element_type=jnp.float32) \
        + bc_ref[...]


_CBLK = 1000


def _cls(degp, accp, y, W_cls, bc2):
    grid = (N // _CBLK,)
    return pl.pallas_call(
        _cls_body,
        grid=grid,
        in_specs=[
            pl.BlockSpec((NCORES, _CBLK, 16), lambda i: (0, i, 0)),
            pl.BlockSpec((NCORES, _CBLK, F), lambda i: (0, i, 0)),
            pl.BlockSpec((_CBLK, F), lambda i: (i, 0)),
            pl.BlockSpec((C, F), lambda i: (0, 0)),
            pl.BlockSpec((1, C), lambda i: (0, 0)),
        ],
        out_specs=pl.BlockSpec((_CBLK, C), lambda i: (i, 0)),
        out_shape=jax.ShapeDtypeStruct((N, C), jnp.float32),
    )(degp, accp, y, W_cls, bc2)


def kernel(x, edge_index, p_topk, init_w, W_ih, W_hh, b_ih, b_hh, W_cls,
           b_cls):
    xp = jnp.zeros((NPAD, F), jnp.float32).at[:N].set(x)
    p2 = p_topk.reshape(1, F)
    b_ih2 = b_ih.reshape(1, 3 * F)
    b_hh2 = b_hh.reshape(1, 3 * F)
    bc2 = b_cls.reshape(1, C)

    W = _evolve(xp, p2, init_w, W_ih, W_hh, b_ih2, b_hh2)
    degp = _sc_degree(edge_index)
    y = _y(degp, xp, W)
    accp = _sc_scatter(y, edge_index)
    return _cls(degp, accp, y, W_cls, bc2)


# speedup vs baseline: 1.2053x; 1.2053x over previous
import functools

import jax
import jax.numpy as jnp
from jax import lax
from jax.experimental import pallas as pl
from jax.experimental.pallas import tpu as pltpu
from jax.experimental.pallas import tpu_sc as plsc

N = 10000
F = 128
E = 320000
C = 2
NPAD = 10240
TRASH = N

NCORES = 2
NSUB = 16
NW = NCORES * NSUB
EPT = E // NW
K = 80
NCH = EPT // K
RPT = NPAD // NSUB


def _evolve_body(xp_ref, p_ref, iw_ref, wih_ref, whh_ref, bih_ref, bhh_ref,
                 w_out_ref, s_scr, vals_scr, xt_scr):
    p = p_ref[...]
    pnorm = jnp.sqrt(jnp.sum(p * p))
    s = lax.dot_general(p, xp_ref[...], (((1,), (1,)), ((), ())),
                        preferred_element_type=jnp.float32)
    col = lax.broadcasted_iota(jnp.int32, (1, NPAD), 1)
    s_scr[...] = jnp.reshape(jnp.where(col < N, s, -jnp.inf), (NPAD // F, F))

    fiota = (lax.broadcasted_iota(jnp.int32, (NPAD // F, F), 0) * F
             + lax.broadcasted_iota(jnp.int32, (NPAD // F, F), 1))
    ninf = jnp.float32(-jnp.inf)

    def sel_body(i, _):
        sv = s_scr[...]
        m = jnp.max(sv)
        idx = jnp.min(jnp.where(sv == m, fiota, NPAD))
        s_scr[...] = jnp.where(fiota == idx, ninf, sv)
        xt_scr[pl.ds(i, 1), :] = xp_ref[pl.ds(idx, 1), :]
        vals_scr[pl.ds(i, 1), :] = jnp.full((1, 1), m, jnp.float32)
        return 0

    lax.fori_loop(0, F, sel_body, 0)

    xt = xt_scr[...] * jnp.tanh(vals_scr[...] / pnorm)
    gi = lax.dot_general(xt, wih_ref[...], (((1,), (1,)), ((), ())),
                         preferred_element_type=jnp.float32) + bih_ref[...]
    gh = lax.dot_general(iw_ref[...], whh_ref[...], (((1,), (1,)), ((), ())),
                         preferred_element_type=jnp.float32) + bhh_ref[...]
    r = jax.nn.sigmoid(gi[:, :F] + gh[:, :F])
    z = jax.nn.sigmoid(gi[:, F:2 * F] + gh[:, F:2 * F])
    ng = jnp.tanh(gi[:, 2 * F:] + r * gh[:, 2 * F:])
    w_out_ref[...] = (1.0 - z) * ng + z * iw_ref[...]


def _evolve(xp, p2, init_w, W_ih, W_hh, b_ih2, b_hh2):
    return pl.pallas_call(
        _evolve_body,
        out_shape=jax.ShapeDtypeStruct((F, F), jnp.float32),
        scratch_shapes=[
            pltpu.VMEM((NPAD // F, F), jnp.float32),
            pltpu.VMEM((F, 1), jnp.float32),
            pltpu.VMEM((F, F), jnp.float32),
        ],
    )(xp, p2, init_w, W_ih, W_hh, b_ih2, b_hh2)


def _deg_body(ei_hbm, out_hbm, acc_sh, srcf, dstf, idxs, ones_v,
              zbuf):
    c = lax.axis_index("c")
    s = lax.axis_index("s")
    wid = s * NCORES + c
    zeros16 = jnp.zeros((16,), jnp.float32)
    ones16 = jnp.ones((16,), jnp.float32)

    def zb(i, _):
        zbuf[i, :] = zeros16
        return 0
    lax.fori_loop(0, RPT, zb, 0)

    def ob(i, _):
        ones_v[i, :] = ones16
        return 0
    lax.fori_loop(0, K, ob, 0)

    pltpu.sync_copy(zbuf, acc_sh.at[pl.ds(s * RPT, RPT)])
    plsc.subcore_barrier()

    base = wid * EPT
    pltpu.sync_copy(ei_hbm.at[0, pl.ds(base, EPT)], srcf)
    pltpu.sync_copy(ei_hbm.at[1, pl.ds(base, EPT)], dstf)

    def remap_chunk(k, _):
        def remap16(j, _):
            off = k * K + j * 16
            sv = srcf[pl.ds(off, 16)]
            dv = dstf[pl.ds(off, 16)]
            idxs[k, pl.ds(j * 16, 16)] = jnp.where(sv == dv, TRASH, dv)
            return 0
        lax.fori_loop(0, K // 16, remap16, 0)
        return 0
    lax.fori_loop(0, NCH, remap_chunk, 0)

    def scat(k, _):
        pltpu.sync_copy(ones_v, acc_sh.at[idxs.at[k]], add=True)
        return 0
    lax.fori_loop(0, NCH, scat, 0)

    plsc.subcore_barrier()
    pltpu.sync_copy(acc_sh.at[pl.ds(s * RPT, RPT)], zbuf)
    pltpu.sync_copy(zbuf, out_hbm.at[c, pl.ds(s * RPT, RPT)])


def _sc_degree(edge_index):
    mesh = plsc.VectorSubcoreMesh(core_axis_name="c", subcore_axis_name="s")
    fn = pl.kernel(
        _deg_body,
        out_type=jax.ShapeDtypeStruct((NCORES, NPAD, 16), jnp.float32),
        mesh=mesh,
        compiler_params=pltpu.CompilerParams(use_tc_tiling_on_sc=False),
        scratch_types=[
            pltpu.VMEM_SHARED((NPAD, 16), jnp.float32),
            pltpu.VMEM((EPT,), jnp.int32),
            pltpu.VMEM((EPT,), jnp.int32),
            pltpu.VMEM((NCH, K), jnp.int32),
            pltpu.VMEM((K, 16), jnp.float32),
            pltpu.VMEM((RPT, 16), jnp.float32),
        ],
    )
    return fn(edge_index)


_BLK = 1024


def _y_body(deg_ref, x_ref, w_ref, y_ref):
    deg = deg_ref[0, :, 0:1] + deg_ref[1, :, 0:1] + 1.0
    dis = 1.0 / jnp.sqrt(deg)
    xw = jnp.dot(x_ref[...], w_ref[...], preferred_element_type=jnp.float32)
    y_ref[...] = dis * xw


def _y(degp, xp, W):
    grid = (NPAD // _BLK,)
    return pl.pallas_call(
        _y_body,
        grid=grid,
        in_specs=[
            pl.BlockSpec((NCORES, _BLK, 16), lambda i: (0, i, 0)),
            pl.BlockSpec((_BLK, F), lambda i: (i, 0)),
            pl.BlockSpec((F, F), lambda i: (0, 0)),
        ],
        out_specs=pl.BlockSpec((_BLK, F), lambda i: (i, 0)),
        out_shape=jax.ShapeDtypeStruct((NPAD, F), jnp.float32),
    )(degp, xp, W)


PH = 5
PEDGE = EPT // PH
PCH = PEDGE // K
NBUF = 3


def _scat_body(y_hbm, ei_hbm, out_hbm, acc_sh, srcf, dstf, idxs,
               rows_v, sem):
    c = lax.axis_index("c")
    s = lax.axis_index("s")
    wid = s * NCORES + c
    zeros16 = jnp.zeros((16,), jnp.float32)

    def zb(i, _):
        for j in range(F // 16):
            rows_v[0, i, pl.ds(j * 16, 16)] = zeros16
        return 0
    lax.fori_loop(0, K, zb, 0)

    for q in range(RPT // K):
        pltpu.sync_copy(rows_v.at[0], acc_sh.at[pl.ds(s * RPT + q * K, K)])
    plsc.subcore_barrier()

    base = wid * EPT

    def phase(ph, _):
        pbase = base + ph * PEDGE
        pltpu.sync_copy(ei_hbm.at[0, pl.ds(pbase, PEDGE)], srcf)
        pltpu.sync_copy(ei_hbm.at[1, pl.ds(pbase, PEDGE)], dstf)

        def remap_chunk(k, _):
            def remap16(j, _):
                off = k * K + j * 16
                sv = srcf[pl.ds(off, 16)]
                dv = dstf[pl.ds(off, 16)]
                idxs[k, pl.ds(j * 16, 16)] = jnp.where(sv == dv, TRASH, dv)
                return 0
            lax.fori_loop(0, K // 16, remap16, 0)
            return 0
        lax.fori_loop(0, PCH, remap_chunk, 0)

        for b in range(NBUF):
            pltpu.async_copy(y_hbm.at[srcf.at[pl.ds(b * K, K)]],
                             rows_v.at[b], sem.at[b])

        def chunk(k, _):
            b = lax.rem(k, NBUF)
            pltpu.make_async_copy(y_hbm.at[srcf.at[pl.ds(k * K, K)]],
                                  rows_v.at[b], sem.at[b]).wait()
            pltpu.sync_copy(rows_v.at[b], acc_sh.at[idxs.at[k]], add=True)

            @pl.when(k + NBUF < PCH)
            def _():
                pltpu.async_copy(
                    y_hbm.at[srcf.at[pl.ds((k + NBUF) * K, K)]],
                    rows_v.at[b], sem.at[b])
            return 0
        lax.fori_loop(0, PCH, chunk, 0)
        return 0
    lax.fori_loop(0, PH, phase, 0)

    plsc.subcore_barrier()
    for q in range(RPT // K):
        r0 = s * RPT + q * K
        pltpu.sync_copy(acc_sh.at[pl.ds(r0, K)], rows_v.at[0])
        pltpu.sync_copy(rows_v.at[0], out_hbm.at[c, pl.ds(r0, K)])


def _sc_scatter(y, edge_index):
    mesh = plsc.VectorSubcoreMesh(core_axis_name="c", subcore_axis_name="s")
    fn = pl.kernel(
        _scat_body,
        out_type=jax.ShapeDtypeStruct((NCORES, NPAD, F), jnp.float32),
        mesh=mesh,
        compiler_params=pltpu.CompilerParams(use_tc_tiling_on_sc=False),
        scratch_types=[
            pltpu.VMEM_SHARED((NPAD, F), jnp.float32),
            pltpu.VMEM((PEDGE,), jnp.int32),
            pltpu.VMEM((PEDGE,), jnp.int32),
            pltpu.VMEM((PCH, K), jnp.int32),
            pltpu.VMEM((NBUF, K, F), jnp.float32),
            pltpu.SemaphoreType.DMA((NBUF,)),
        ],
    )
    return fn(y, edge_index)


def _cls_body(deg_ref, acc_ref, y_ref, wc_ref, bc_ref, o_ref):
    deg = deg_ref[0, :, 0:1] + deg_ref[1, :, 0:1] + 1.0
    dis = 1.0 / jnp.sqrt(deg)
    h = dis * (acc_ref[0] + acc_ref[1] + y_ref[...])
    h = jnp.maximum(h, 0.0)
    o_ref[...] = lax.dot_general(h, wc_ref[...], (((1,), (1,)), ((), ())),
                                 preferred_element_type=jnp.float32) \
        + bc_ref[...]


_CBLK = 1000


def _cls(degp, accp, y, W_cls, bc2):
    grid = (N // _CBLK,)
    return pl.pallas_call(
        _cls_body,
        grid=grid,
        in_specs=[
            pl.BlockSpec((NCORES, _CBLK, 16), lambda i: (0, i, 0)),
            pl.BlockSpec((NCORES, _CBLK, F), lambda i: (0, i, 0)),
            pl.BlockSpec((_CBLK, F), lambda i: (i, 0)),
            pl.BlockSpec((C, F), lambda i: (0, 0)),
            pl.BlockSpec((1, C), lambda i: (0, 0)),
        ],
        out_specs=pl.BlockSpec((_CBLK, C), lambda i: (i, 0)),
        out_shape=jax.ShapeDtypeStruct((N, C), jnp.float32),
    )(degp, accp, y, W_cls, bc2)


def kernel(x, edge_index, p_topk, init_w, W_ih, W_hh, b_ih, b_hh, W_cls,
           b_cls):
    xp = jnp.zeros((NPAD, F), jnp.float32).at[:N].set(x)
    p2 = p_topk.reshape(1, F)
    b_ih2 = b_ih.reshape(1, 3 * F)
    b_hh2 = b_hh.reshape(1, 3 * F)
    bc2 = b_cls.reshape(1, C)

    W = _evolve(xp, p2, init_w, W_ih, W_hh, b_ih2, b_hh2)
    degp = _sc_degree(edge_index)
    y = _y(degp, xp, W)
    accp = _sc_scatter(y, edge_index)
    return _cls(degp, accp, y, W_cls, bc2)
